# Initial kernel scaffold; baseline (speedup 1.0000x reference)
import jax
import jax.numpy as jnp
from jax.experimental import pallas as pl


def kernel(input_bits, connections, memory):
    raise NotImplementedError("write your pallas kernel here")


# speedup vs baseline: 2.0031x; 2.0031x over previous
import jax
import jax.numpy as jnp
from jax import lax
from jax.experimental import pallas as pl
from jax.experimental.pallas import tpu as pltpu
from jax.experimental.pallas import tpu_sc as plsc

B = 1024
J = 2048
N = 4096
K = 12
NC = 2
NS = 16
LANES = 16

N_PER_SC = N // NC
B_PER_TILE = B // NS
N_PER_TILE = N_PER_SC // NS
CONN_BLK = 256
N_CHUNK = 16
GROUPS = B // (4 * LANES)


def _sc_body(t3_hbm, conn_hbm, mem_hbm, out_hbm, addr_sh):
    c = lax.axis_index("c")
    s = lax.axis_index("s")
    lane = lax.iota(jnp.int32, LANES)

    def phase1(inp_v, conn_v, accbuf):
        pltpu.sync_copy(t3_hbm.at[s], inp_v)

        def blk_body(blk, _):
            n0 = c * N_PER_SC + blk * CONN_BLK
            pltpu.sync_copy(conn_hbm.at[pl.ds(n0, CONN_BLK), :], conn_v)

            def n_body(nn, _):
                hi = jnp.zeros((LANES,), jnp.int32)
                lo = jnp.zeros((LANES,), jnp.int32)
                cvec = conn_v[nn, pl.ds(0, LANES)]
                for k in range(K):
                    ck = cvec[k]
                    wv = plsc.load_gather(inp_v, [lane + ck * 16])
                    if k < 6:
                        hi = hi + (wv << (5 - k))
                    else:
                        lo = lo + (wv << (11 - k))
                accbuf[nn, pl.ds(0, LANES)] = hi
                accbuf[nn, pl.ds(LANES, LANES)] = lo
                return 0

            lax.fori_loop(0, CONN_BLK, n_body, 0)
            pltpu.sync_copy(accbuf,
                            addr_sh.at[s, pl.ds(blk * CONN_BLK, CONN_BLK), :])
            return 0

        lax.fori_loop(0, N_PER_SC // CONN_BLK, blk_body, 0)

    pl.run_scoped(
        phase1,
        pltpu.VMEM((J * 16,), jnp.int32),
        pltpu.VMEM((CONN_BLK, LANES), jnp.int32),
        pltpu.VMEM((CONN_BLK, 2 * LANES), jnp.int32),
    )
    plsc.subcore_barrier()

    nl0 = s * N_PER_TILE
    ns0 = c * N_PER_SC + nl0

    def phase2(mem_v, addr_v, out_v):
        def chunk_body(j, _):
            r0 = ns0 + j * N_CHUNK
            pltpu.sync_copy(mem_hbm.at[pl.ds(r0, N_CHUNK), :], mem_v)
            for g in range(GROUPS):
                pltpu.sync_copy(
                    addr_sh.at[g, pl.ds(nl0 + j * N_CHUNK, N_CHUNK), :],
                    addr_v.at[g])

            def i_body(i, _):
                row = jnp.zeros((LANES,), jnp.int32) + i

                def g_body(g, _):
                    hi = addr_v[g, i, pl.ds(0, LANES)]
                    lo = addr_v[g, i, pl.ds(LANES, LANES)]
                    out_w = jnp.zeros((LANES,), jnp.int32)
                    for bi in range(4):
                        h = (hi >> (8 * bi)) & 63
                        l = (lo >> (8 * bi)) & 63
                        a = (h << 6) | l
                        wv = plsc.load_gather(mem_v, [row, a >> 2])
                        byte = (wv >> ((a & 3) << 3)) & 255
                        r = (byte == 2).astype(jnp.int32)
                        out_w = out_w | (r << (8 * bi))
                    out_v[i, pl.ds(g * LANES, LANES)] = out_w
                    return 0

                lax.fori_loop(0, GROUPS, g_body, 0)
                return 0

            lax.fori_loop(0, N_CHUNK, i_body, 0)
            pltpu.sync_copy(out_v, out_hbm.at[pl.ds(r0, N_CHUNK), :])
            return 0

        lax.fori_loop(0, N_PER_TILE // N_CHUNK, chunk_body, 0)

    pl.run_scoped(
        phase2,
        pltpu.VMEM((N_CHUNK, 1024), jnp.int32),
        pltpu.VMEM((GROUPS, N_CHUNK, 2 * LANES), jnp.int32),
        pltpu.VMEM((N_CHUNK, B // 4), jnp.int32),
    )


def _sc_call(t3w, conn, mem32):
    mesh = plsc.VectorSubcoreMesh(core_axis_name="c", subcore_axis_name="s")
    return pl.kernel(
        _sc_body,
        out_type=jax.ShapeDtypeStruct((N, B // 4), jnp.int32),
        mesh=mesh,
        compiler_params=pltpu.CompilerParams(
            needs_layout_passes=False,
            use_tc_tiling_on_sc=False,
        ),
        scratch_types=[
            pltpu.VMEM_SHARED((NS, N_PER_SC, 2 * LANES), jnp.int32),
        ],
    )(t3w, conn, mem32)


def kernel(input_bits, connections, memory):
    tb = input_bits.astype(jnp.uint8).T
    t3 = tb.reshape(J, NS, B_PER_TILE).transpose(1, 0, 2)
    t3w = lax.bitcast_convert_type(
        t3.reshape(NS, J, LANES, 4), jnp.int32).reshape(NS, J * LANES)
    mem32 = lax.bitcast_convert_type(
        memory.reshape(N, 1024, 4), jnp.int32)
    conn_p = jnp.pad(connections, ((0, 0), (0, LANES - K)))
    outT = _sc_call(t3w, conn_p, mem32)
    out_u8 = lax.bitcast_convert_type(outT, jnp.uint8).reshape(N, B)
    return out_u8.T.astype(jnp.bool_)
